# Initial kernel scaffold; baseline (speedup 1.0000x reference)
import jax
import jax.numpy as jnp
from jax.experimental import pallas as pl


def kernel(x, edge_index, edge_weight, W1, b1, W2, b2, W3, b3, W4, b4, W5, b5, Wf1, bf1, Wf2, bf2, Wf3, bf3):
    raise NotImplementedError("write your pallas kernel here")


# speedup vs baseline: 13.9914x; 13.9914x over previous
import functools

import jax
import jax.numpy as jnp
from jax import lax
from jax.experimental import pallas as pl
from jax.experimental.pallas import tpu as pltpu
from jax.experimental.pallas import tpu_sc as plsc

N = 50000
NP = 51200
RPN = NP // 16
E = 1600000
EP = 1605632
ROWS = EP // 128
NC, NS = 2, 16
NW = NC * NS
RPT = ROWS // NW
BLK = 8
NBLK = RPT // BLK
F = 32

_mesh = plsc.VectorSubcoreMesh(core_axis_name="c", subcore_axis_name="s")
_sc_params = pltpu.CompilerParams(needs_layout_passes=False,
                                 use_tc_tiling_on_sc=False)


def _wid():
    return lax.axis_index("c") * NS + lax.axis_index("s")


@functools.partial(
    pl.kernel,
    out_type=jax.ShapeDtypeStruct((NC * NP,), jnp.float32),
    mesh=_mesh,
    compiler_params=_sc_params,
    scratch_types=[
        pltpu.VMEM((BLK, 128), jnp.int32),
        pltpu.VMEM((BLK, 128), jnp.float32),
        pltpu.VMEM_SHARED((NP,), jnp.float32),
    ],
)
def _sc_degree(srcM, wM, zeros1, out, sbuf, wbuf, acc):
    cid = lax.axis_index("c")
    sid = lax.axis_index("s")
    wid = _wid()
    pltpu.sync_copy(zeros1, acc.at[pl.ds(sid * RPN, RPN)])
    plsc.subcore_barrier()

    def blk(i, _):
        r0 = wid * RPT + i * BLK
        pltpu.sync_copy(srcM.at[pl.ds(r0, BLK)], sbuf)
        pltpu.sync_copy(wM.at[pl.ds(r0, BLK)], wbuf)
        for j in range(BLK):
            pltpu.sync_copy(wbuf.at[j], acc.at[sbuf.at[j]], add=True)
        return 0

    lax.fori_loop(0, NBLK, blk, 0)
    plsc.subcore_barrier()
    pltpu.sync_copy(acc.at[pl.ds(sid * RPN, RPN)],
                    out.at[pl.ds(cid * NP + sid * RPN, RPN)])


@functools.partial(
    pl.kernel,
    out_type=jax.ShapeDtypeStruct((ROWS, 128), jnp.float32),
    mesh=_mesh,
    compiler_params=_sc_params,
    scratch_types=[
        pltpu.VMEM((BLK, 128), jnp.int32),
        pltpu.VMEM((BLK, 128), jnp.int32),
        pltpu.VMEM((BLK, 128), jnp.float32),
        pltpu.VMEM((BLK, 128), jnp.float32),
        pltpu.VMEM((NP,), jnp.float32),
    ],
)
def _sc_norm(srcM, dstM, wM, dinv, out, sbuf, dbuf, wbuf, obuf, tbuf):
    wid = _wid()
    pltpu.sync_copy(dinv, tbuf)

    def blk(i, _):
        r0 = wid * RPT + i * BLK
        pltpu.sync_copy(srcM.at[pl.ds(r0, BLK)], sbuf)
        pltpu.sync_copy(dstM.at[pl.ds(r0, BLK)], dbuf)
        pltpu.sync_copy(wM.at[pl.ds(r0, BLK)], wbuf)
        for j in range(BLK):
            for g in range(8):
                sl = pl.ds(g * 16, 16)
                ds_ = plsc.load_gather(tbuf, [sbuf[j, sl]])
                dd = plsc.load_gather(tbuf, [dbuf[j, sl]])
                obuf[j, sl] = -(ds_ * wbuf[j, sl] * dd)
        pltpu.sync_copy(obuf, out.at[pl.ds(r0, BLK)])
        return 0

    lax.fori_loop(0, NBLK, blk, 0)


@functools.partial(
    pl.kernel,
    out_type=jax.ShapeDtypeStruct((NC * NP,), jnp.float32),
    mesh=_mesh,
    compiler_params=_sc_params,
    scratch_types=[
        pltpu.VMEM((BLK, 128), jnp.int32),
        pltpu.VMEM((BLK, 128), jnp.int32),
        pltpu.VMEM((BLK, 128), jnp.float32),
        pltpu.VMEM((BLK, 128), jnp.float32),
        pltpu.VMEM((NP,), jnp.float32),
        pltpu.VMEM_SHARED((NP,), jnp.float32),
    ],
)
def _sc_lmv1(t, srcM, dstM, nwM, zeros1, out, sbuf, dbuf, wbuf, vbuf, tbuf,
             acc):
    cid = lax.axis_index("c")
    sid = lax.axis_index("s")
    wid = _wid()
    pltpu.sync_copy(zeros1, acc.at[pl.ds(sid * RPN, RPN)])
    pltpu.sync_copy(t, tbuf)
    plsc.subcore_barrier()

    def blk(i, _):
        r0 = wid * RPT + i * BLK
        pltpu.sync_copy(srcM.at[pl.ds(r0, BLK)], sbuf)
        pltpu.sync_copy(dstM.at[pl.ds(r0, BLK)], dbuf)
        pltpu.sync_copy(nwM.at[pl.ds(r0, BLK)], wbuf)
        for j in range(BLK):
            for g in range(8):
                sl = pl.ds(g * 16, 16)
                vals = plsc.load_gather(tbuf, [sbuf[j, sl]])
                vbuf[j, sl] = vals * wbuf[j, sl]
        for j in range(BLK):
            pltpu.sync_copy(vbuf.at[j], acc.at[dbuf.at[j]], add=True)
        return 0

    lax.fori_loop(0, NBLK, blk, 0)
    plsc.subcore_barrier()
    pltpu.sync_copy(acc.at[pl.ds(sid * RPN, RPN)],
                    out.at[pl.ds(cid * NP + sid * RPN, RPN)])


@functools.partial(
    pl.kernel,
    out_type=jax.ShapeDtypeStruct((NC, 2, NP, 16), jnp.float32),
    mesh=_mesh,
    compiler_params=_sc_params,
    scratch_types=[
        pltpu.VMEM((BLK, 128), jnp.int32),
        pltpu.VMEM((BLK, 128), jnp.int32),
        pltpu.VMEM((BLK, 128), jnp.float32),
        pltpu.VMEM((BLK * 128, 16), jnp.float32),
        pltpu.VMEM_SHARED((NP, 16), jnp.float32),
        pltpu.SemaphoreType.DMA,
    ],
)
def _sc_lmv32(t0h, t1h, srcM, dstM, nwM, zeros16, out, sbuf, dbuf, wbuf,
              rows, acc, sem):
    cid = lax.axis_index("c")
    sid = lax.axis_index("s")
    wid = _wid()
    for p, th in ((0, t0h), (1, t1h)):
        pltpu.sync_copy(zeros16, acc.at[pl.ds(sid * RPN, RPN), :])
        plsc.subcore_barrier()

        def blk(i, _):
            r0 = wid * RPT + i * BLK
            pltpu.sync_copy(srcM.at[pl.ds(r0, BLK)], sbuf)
            pltpu.sync_copy(dstM.at[pl.ds(r0, BLK)], dbuf)
            pltpu.sync_copy(nwM.at[pl.ds(r0, BLK)], wbuf)
            descs = [
                pltpu.async_copy(th.at[sbuf.at[j]],
                                 rows.at[pl.ds(j * 128, 128)], sem)
                for j in range(BLK)
            ]
            for d in descs:
                d.wait()

            @plsc.parallel_loop(0, BLK * 8, unroll=2)
            def mul(g):
                wv = wbuf[g // 8, pl.ds((g % 8) * 16, 16)]
                base = g * 16
                for l in range(16):
                    e = base + l
                    rows[e, pl.ds(0, 16)] = rows[e, pl.ds(0, 16)] * wv[l]

            for j in range(BLK):
                pltpu.sync_copy(rows.at[pl.ds(j * 128, 128)],
                                acc.at[dbuf.at[j]], add=True)
            return 0

        lax.fori_loop(0, NBLK, blk, 0)
        plsc.subcore_barrier()
        pltpu.sync_copy(acc.at[pl.ds(sid * RPN, RPN), :],
                        out.at[cid, p, pl.ds(sid * RPN, RPN), :])


def _tc_dinv(degp):
    def body(p_ref, o_ref):
        deg = p_ref[0, :] + p_ref[1, :]
        pos = deg > 0.0
        safe = jnp.where(pos, deg, 1.0)
        o_ref[...] = jnp.where(pos, lax.rsqrt(safe), 0.0)

    return pl.pallas_call(
        body, out_shape=jax.ShapeDtypeStruct((NP,), jnp.float32),
    )(degp)


def _tc_comb1(p, prev, alpha, beta):
    def body(p_ref, v_ref, o_ref):
        s = p_ref[0, :] + p_ref[1, :]
        o_ref[...] = alpha * s + beta * v_ref[...]

    return pl.pallas_call(
        body, out_shape=jax.ShapeDtypeStruct((NP,), jnp.float32),
    )(p, prev)


def _tc_comb32(p, prev, alpha, beta):
    blk = 3200

    def body(p_ref, v_ref, o_ref):
        o_ref[0] = alpha * (p_ref[0, 0] + p_ref[1, 0]) + beta * v_ref[0]
        o_ref[1] = alpha * (p_ref[0, 1] + p_ref[1, 1]) + beta * v_ref[1]

    return pl.pallas_call(
        body,
        grid=(NP // blk,),
        in_specs=[
            pl.BlockSpec((NC, 2, blk, 16), lambda i: (0, 0, i, 0)),
            pl.BlockSpec((2, blk, 16), lambda i: (0, i, 0)),
        ],
        out_specs=pl.BlockSpec((2, blk, 16), lambda i: (0, i, 0)),
        out_shape=jax.ShapeDtypeStruct((2, NP, 16), jnp.float32),
    )(p, prev)


def _tc_layer1(ts, W, b):
    blk = 3200

    def body(t0, t1, t2, t3, w_ref, b_ref, o_ref):
        w = w_ref[:, 0, :]
        acc = t0[...] * w[0][None, :]
        acc += t1[...] * w[1][None, :]
        acc += t2[...] * w[2][None, :]
        acc += t3[...] * w[3][None, :]
        acc = jnp.maximum(acc + b_ref[...][None, :], 0.0)
        o_ref[0] = acc[:, :16]
        o_ref[1] = acc[:, 16:]

    tspec = pl.BlockSpec((blk, 1), lambda i: (i, 0))
    return pl.pallas_call(
        body,
        grid=(NP // blk,),
        in_specs=[tspec, tspec, tspec, tspec,
                  pl.BlockSpec((4, 1, F), lambda i: (0, 0, 0)),
                  pl.BlockSpec((F,), lambda i: (0,))],
        out_specs=pl.BlockSpec((2, blk, 16), lambda i: (0, i, 0)),
        out_shape=jax.ShapeDtypeStruct((2, NP, 16), jnp.float32),
    )(*[t.reshape(NP, 1) for t in ts], W, b)


def _tc_layer(ts, W, b, fo):
    blk = 3200
    halves_out = fo == F

    def body(t0, t1, t2, t3, w_ref, b_ref, o_ref):
        acc = None
        for t_ref, k in ((t0, 0), (t1, 1), (t2, 2), (t3, 3)):
            t = jnp.concatenate([t_ref[0], t_ref[1]], axis=-1)
            d = jnp.dot(t, w_ref[k], preferred_element_type=jnp.float32)
            acc = d if acc is None else acc + d
        acc = jnp.maximum(acc + b_ref[...][None, :], 0.0)
        if halves_out:
            o_ref[0] = acc[:, :16]
            o_ref[1] = acc[:, 16:]
        else:
            o_ref[...] = acc

    tspec = pl.BlockSpec((2, blk, 16), lambda i: (0, i, 0))
    if halves_out:
        out_spec = pl.BlockSpec((2, blk, 16), lambda i: (0, i, 0))
        out_shape = jax.ShapeDtypeStruct((2, NP, 16), jnp.float32)
    else:
        out_spec = pl.BlockSpec((blk, fo), lambda i: (i, 0))
        out_shape = jax.ShapeDtypeStruct((NP, fo), jnp.float32)
    return pl.pallas_call(
        body,
        grid=(NP // blk,),
        in_specs=[tspec, tspec, tspec, tspec,
                  pl.BlockSpec((4, F, fo), lambda i: (0, 0, 0)),
                  pl.BlockSpec((fo,), lambda i: (0,))],
        out_specs=out_spec,
        out_shape=out_shape,
    )(*ts, W, b)


def _tc_fc(xf, Wf1, bf1, Wf2, bf2, Wf3, bf3):
    kb = 2000
    nk = 200000 // kb

    def body(x_ref, w1_ref, b1_ref, w2_ref, b2_ref, w3_ref, b3_ref, o_ref,
             acc):
        i = pl.program_id(0)

        @pl.when(i == 0)
        def _():
            acc[...] = jnp.zeros_like(acc)

        acc[...] += jnp.dot(x_ref[0], w1_ref[0],
                            preferred_element_type=jnp.float32)

        @pl.when(i == nk - 1)
        def _():
            z = acc[...] + b1_ref[...][None, :]
            z = jnp.dot(z, w2_ref[...],
                        preferred_element_type=jnp.float32) + b2_ref[...][None, :]
            z = jnp.dot(z, w3_ref[...],
                        preferred_element_type=jnp.float32) + b3_ref[...][None, :]
            o_ref[...] = z

    return pl.pallas_call(
        body,
        grid=(nk,),
        in_specs=[
            pl.BlockSpec((1, 1, kb), lambda i: (i, 0, 0)),
            pl.BlockSpec((1, kb, 128), lambda i: (i, 0, 0)),
            pl.BlockSpec((128,), lambda i: (0,)),
            pl.BlockSpec((128, 128), lambda i: (0, 0)),
            pl.BlockSpec((128,), lambda i: (0,)),
            pl.BlockSpec((128, 9), lambda i: (0, 0)),
            pl.BlockSpec((9,), lambda i: (0,)),
        ],
        out_specs=pl.BlockSpec((1, 9), lambda i: (0, 0)),
        out_shape=jax.ShapeDtypeStruct((1, 9), jnp.float32),
        scratch_shapes=[pltpu.VMEM((1, 128), jnp.float32)],
    )(xf.reshape(nk, 1, kb), Wf1.reshape(nk, kb, 128), bf1, Wf2, bf2, Wf3,
      bf3)


def kernel(x, edge_index, edge_weight, W1, b1, W2, b2, W3, b3, W4, b4, W5, b5,
           Wf1, bf1, Wf2, bf2, Wf3, bf3):
    ipad = jnp.zeros((EP - E,), jnp.int32)
    fpad = jnp.zeros((EP - E,), jnp.float32)
    srcM = jnp.concatenate([edge_index[0], ipad]).reshape(ROWS, 128)
    dstM = jnp.concatenate([edge_index[1], ipad]).reshape(ROWS, 128)
    wM = jnp.concatenate([edge_weight, fpad]).reshape(ROWS, 128)
    zeros1 = jnp.zeros((RPN,), jnp.float32)
    zeros16 = jnp.zeros((RPN, 16), jnp.float32)
    xp = jnp.concatenate([x[:, 0], jnp.zeros((NP - N,), jnp.float32)])

    degp = _sc_degree(srcM, wM, zeros1).reshape(NC, NP)
    dinv = _tc_dinv(degp)
    nwM = _sc_norm(srcM, dstM, wM, dinv)

    def lmv1(t):
        return _sc_lmv1(t, srcM, dstM, nwM, zeros1).reshape(NC, NP)

    t0 = xp
    t1 = _tc_comb1(lmv1(t0), t0, 1.0, 0.0)
    t2 = _tc_comb1(lmv1(t1), t0, 2.0, -1.0)
    t3 = _tc_comb1(lmv1(t2), t1, 2.0, -1.0)
    h = _tc_layer1([t0, t1, t2, t3], W1, b1)

    def lmv32(t):
        return _sc_lmv32(t[0], t[1], srcM, dstM, nwM, zeros16)

    for W, b, fo in ((W2, b2, F), (W3, b3, F), (W4, b4, F), (W5, b5, 4)):
        t0 = h
        t1 = _tc_comb32(lmv32(t0), t0, 1.0, 0.0)
        t2 = _tc_comb32(lmv32(t1), t0, 2.0, -1.0)
        t3 = _tc_comb32(lmv32(t2), t1, 2.0, -1.0)
        h = _tc_layer([t0, t1, t2, t3], W, b, fo)

    xf = h[:N, :].reshape(1, N * 4)
    return _tc_fc(xf, Wf1, bf1, Wf2, bf2, Wf3, bf3)


# speedup vs baseline: 14.7802x; 1.0564x over previous
import functools

import jax
import jax.numpy as jnp
from jax import lax
from jax.experimental import pallas as pl
from jax.experimental.pallas import tpu as pltpu
from jax.experimental.pallas import tpu_sc as plsc

N = 50000
NP = 51200
RPN = NP // 16
E = 1600000
EP = 1605632
NC, NS = 2, 16
NW = NC * NS
EPT = EP // NW
CH = 1024
NBLK = EPT // CH
F = 32

_mesh = plsc.VectorSubcoreMesh(core_axis_name="c", subcore_axis_name="s")
_sc_params = pltpu.CompilerParams(needs_layout_passes=False,
                                 use_tc_tiling_on_sc=False)


def _wid():
    return lax.axis_index("c") * NS + lax.axis_index("s")


@functools.partial(
    pl.kernel,
    out_type=jax.ShapeDtypeStruct((NC * NP,), jnp.float32),
    mesh=_mesh,
    compiler_params=_sc_params,
    scratch_types=[
        pltpu.VMEM((CH,), jnp.int32),
        pltpu.VMEM((CH,), jnp.float32),
        pltpu.VMEM_SHARED((NP,), jnp.float32),
    ],
)
def _sc_degree(src1, w1, zeros1, out, sbuf, wbuf, acc):
    cid = lax.axis_index("c")
    sid = lax.axis_index("s")
    wid = _wid()
    pltpu.sync_copy(zeros1, acc.at[pl.ds(sid * RPN, RPN)])
    plsc.subcore_barrier()

    def blk(i, _):
        e0 = wid * EPT + i * CH
        pltpu.sync_copy(src1.at[pl.ds(e0, CH)], sbuf)
        pltpu.sync_copy(w1.at[pl.ds(e0, CH)], wbuf)
        pltpu.sync_copy(wbuf, acc.at[sbuf], add=True)
        return 0

    lax.fori_loop(0, NBLK, blk, 0)
    plsc.subcore_barrier()
    pltpu.sync_copy(acc.at[pl.ds(sid * RPN, RPN)],
                    out.at[pl.ds(cid * NP + sid * RPN, RPN)])


@functools.partial(
    pl.kernel,
    out_type=jax.ShapeDtypeStruct((EP,), jnp.float32),
    mesh=_mesh,
    compiler_params=_sc_params,
    scratch_types=[
        pltpu.VMEM((CH,), jnp.int32),
        pltpu.VMEM((CH,), jnp.int32),
        pltpu.VMEM((CH,), jnp.float32),
        pltpu.VMEM((CH,), jnp.float32),
        pltpu.VMEM((NP,), jnp.float32),
    ],
)
def _sc_norm(src1, dst1, w1, dinv, out, sbuf, dbuf, wbuf, obuf, tbuf):
    wid = _wid()
    pltpu.sync_copy(dinv, tbuf)

    def blk(i, _):
        e0 = wid * EPT + i * CH
        pltpu.sync_copy(src1.at[pl.ds(e0, CH)], sbuf)
        pltpu.sync_copy(dst1.at[pl.ds(e0, CH)], dbuf)
        pltpu.sync_copy(w1.at[pl.ds(e0, CH)], wbuf)
        for g in range(CH // 16):
            sl = pl.ds(g * 16, 16)
            ds_ = plsc.load_gather(tbuf, [sbuf[sl]])
            dd = plsc.load_gather(tbuf, [dbuf[sl]])
            obuf[sl] = -(ds_ * wbuf[sl] * dd)
        pltpu.sync_copy(obuf, out.at[pl.ds(e0, CH)])
        return 0

    lax.fori_loop(0, NBLK, blk, 0)


@functools.partial(
    pl.kernel,
    out_type=jax.ShapeDtypeStruct((NC * NP,), jnp.float32),
    mesh=_mesh,
    compiler_params=_sc_params,
    scratch_types=[
        pltpu.VMEM((CH,), jnp.int32),
        pltpu.VMEM((CH,), jnp.int32),
        pltpu.VMEM((CH,), jnp.float32),
        pltpu.VMEM((CH,), jnp.float32),
        pltpu.VMEM((NP,), jnp.float32),
        pltpu.VMEM_SHARED((NP,), jnp.float32),
    ],
)
def _sc_lmv1(t, src1, dst1, nw1, zeros1, out, sbuf, dbuf, wbuf, vbuf, tbuf,
             acc):
    cid = lax.axis_index("c")
    sid = lax.axis_index("s")
    wid = _wid()
    pltpu.sync_copy(zeros1, acc.at[pl.ds(sid * RPN, RPN)])
    pltpu.sync_copy(t, tbuf)
    plsc.subcore_barrier()

    def blk(i, _):
        e0 = wid * EPT + i * CH
        pltpu.sync_copy(src1.at[pl.ds(e0, CH)], sbuf)
        pltpu.sync_copy(dst1.at[pl.ds(e0, CH)], dbuf)
        pltpu.sync_copy(nw1.at[pl.ds(e0, CH)], wbuf)
        for g in range(CH // 16):
            sl = pl.ds(g * 16, 16)
            vals = plsc.load_gather(tbuf, [sbuf[sl]])
            vbuf[sl] = vals * wbuf[sl]
        pltpu.sync_copy(vbuf, acc.at[dbuf], add=True)
        return 0

    lax.fori_loop(0, NBLK, blk, 0)
    plsc.subcore_barrier()
    pltpu.sync_copy(acc.at[pl.ds(sid * RPN, RPN)],
                    out.at[pl.ds(cid * NP + sid * RPN, RPN)])


@functools.partial(
    pl.kernel,
    out_type=jax.ShapeDtypeStruct((NC, 2, NP, 16), jnp.float32),
    mesh=_mesh,
    compiler_params=_sc_params,
    scratch_types=[
        pltpu.VMEM((CH,), jnp.int32),
        pltpu.VMEM((CH,), jnp.int32),
        pltpu.VMEM((CH,), jnp.float32),
        pltpu.VMEM((CH, 16), jnp.float32),
        pltpu.VMEM_SHARED((NP, 16), jnp.float32),
        pltpu.SemaphoreType.DMA,
    ],
)
def _sc_lmv32(t0h, t1h, src1, dst1, nw1, zeros16, out, sbuf, dbuf, wbuf,
              rows, acc, sem):
    cid = lax.axis_index("c")
    sid = lax.axis_index("s")
    wid = _wid()
    for p, th in ((0, t0h), (1, t1h)):
        pltpu.sync_copy(zeros16, acc.at[pl.ds(sid * RPN, RPN), :])
        plsc.subcore_barrier()

        def blk(i, _):
            e0 = wid * EPT + i * CH
            pltpu.sync_copy(src1.at[pl.ds(e0, CH)], sbuf)
            pltpu.sync_copy(dst1.at[pl.ds(e0, CH)], dbuf)
            pltpu.sync_copy(nw1.at[pl.ds(e0, CH)], wbuf)
            pltpu.async_copy(th.at[sbuf], rows, sem).wait()

            @plsc.parallel_loop(0, CH // 16, unroll=2)
            def mul(g):
                wv = wbuf[pl.ds(g * 16, 16)]
                base = g * 16
                for l in range(16):
                    e = base + l
                    rows[e, pl.ds(0, 16)] = rows[e, pl.ds(0, 16)] * wv[l]

            pltpu.sync_copy(rows, acc.at[dbuf], add=True)
            return 0

        lax.fori_loop(0, NBLK, blk, 0)
        plsc.subcore_barrier()
        pltpu.sync_copy(acc.at[pl.ds(sid * RPN, RPN), :],
                        out.at[cid, p, pl.ds(sid * RPN, RPN), :])


def _tc_dinv(degp):
    def body(p_ref, o_ref):
        deg = p_ref[0, :] + p_ref[1, :]
        pos = deg > 0.0
        safe = jnp.where(pos, deg, 1.0)
        o_ref[...] = jnp.where(pos, lax.rsqrt(safe), 0.0)

    return pl.pallas_call(
        body, out_shape=jax.ShapeDtypeStruct((NP,), jnp.float32),
    )(degp)


def _tc_comb1(p, prev, alpha, beta):
    def body(p_ref, v_ref, o_ref):
        s = p_ref[0, :] + p_ref[1, :]
        o_ref[...] = alpha * s + beta * v_ref[...]

    return pl.pallas_call(
        body, out_shape=jax.ShapeDtypeStruct((NP,), jnp.float32),
    )(p, prev)


def _tc_comb32(p, prev, alpha, beta):
    blk = 3200

    def body(p_ref, v_ref, o_ref):
        o_ref[0] = alpha * (p_ref[0, 0] + p_ref[1, 0]) + beta * v_ref[0]
        o_ref[1] = alpha * (p_ref[0, 1] + p_ref[1, 1]) + beta * v_ref[1]

    return pl.pallas_call(
        body,
        grid=(NP // blk,),
        in_specs=[
            pl.BlockSpec((NC, 2, blk, 16), lambda i: (0, 0, i, 0)),
            pl.BlockSpec((2, blk, 16), lambda i: (0, i, 0)),
        ],
        out_specs=pl.BlockSpec((2, blk, 16), lambda i: (0, i, 0)),
        out_shape=jax.ShapeDtypeStruct((2, NP, 16), jnp.float32),
    )(p, prev)


def _tc_layer1(ts, W, b):
    blk = 3200

    def body(t0, t1, t2, t3, w_ref, b_ref, o_ref):
        w = w_ref[:, 0, :]
        acc = t0[...] * w[0][None, :]
        acc += t1[...] * w[1][None, :]
        acc += t2[...] * w[2][None, :]
        acc += t3[...] * w[3][None, :]
        acc = jnp.maximum(acc + b_ref[...][None, :], 0.0)
        o_ref[0] = acc[:, :16]
        o_ref[1] = acc[:, 16:]

    tspec = pl.BlockSpec((blk, 1), lambda i: (i, 0))
    return pl.pallas_call(
        body,
        grid=(NP // blk,),
        in_specs=[tspec, tspec, tspec, tspec,
                  pl.BlockSpec((4, 1, F), lambda i: (0, 0, 0)),
                  pl.BlockSpec((F,), lambda i: (0,))],
        out_specs=pl.BlockSpec((2, blk, 16), lambda i: (0, i, 0)),
        out_shape=jax.ShapeDtypeStruct((2, NP, 16), jnp.float32),
    )(*[t.reshape(NP, 1) for t in ts], W, b)


def _tc_layer(ts, W, b, fo):
    blk = 3200
    halves_out = fo == F

    def body(t0, t1, t2, t3, w_ref, b_ref, o_ref):
        acc = None
        for t_ref, k in ((t0, 0), (t1, 1), (t2, 2), (t3, 3)):
            t = jnp.concatenate([t_ref[0], t_ref[1]], axis=-1)
            d = jnp.dot(t, w_ref[k], preferred_element_type=jnp.float32)
            acc = d if acc is None else acc + d
        acc = jnp.maximum(acc + b_ref[...][None, :], 0.0)
        if halves_out:
            o_ref[0] = acc[:, :16]
            o_ref[1] = acc[:, 16:]
        else:
            o_ref[...] = acc

    tspec = pl.BlockSpec((2, blk, 16), lambda i: (0, i, 0))
    if halves_out:
        out_spec = pl.BlockSpec((2, blk, 16), lambda i: (0, i, 0))
        out_shape = jax.ShapeDtypeStruct((2, NP, 16), jnp.float32)
    else:
        out_spec = pl.BlockSpec((blk, fo), lambda i: (i, 0))
        out_shape = jax.ShapeDtypeStruct((NP, fo), jnp.float32)
    return pl.pallas_call(
        body,
        grid=(NP // blk,),
        in_specs=[tspec, tspec, tspec, tspec,
                  pl.BlockSpec((4, F, fo), lambda i: (0, 0, 0)),
                  pl.BlockSpec((fo,), lambda i: (0,))],
        out_specs=out_spec,
        out_shape=out_shape,
    )(*ts, W, b)


def _tc_fc(xf, Wf1, bf1, Wf2, bf2, Wf3, bf3):
    kb = 2000
    nk = 200000 // kb

    def body(x_ref, w1_ref, b1_ref, w2_ref, b2_ref, w3_ref, b3_ref, o_ref,
             acc):
        i = pl.program_id(0)

        @pl.when(i == 0)
        def _():
            acc[...] = jnp.zeros_like(acc)

        acc[...] += jnp.dot(x_ref[0], w1_ref[0],
                            preferred_element_type=jnp.float32)

        @pl.when(i == nk - 1)
        def _():
            z = acc[...] + b1_ref[...][None, :]
            z = jnp.dot(z, w2_ref[...],
                        preferred_element_type=jnp.float32) + b2_ref[...][None, :]
            z = jnp.dot(z, w3_ref[...],
                        preferred_element_type=jnp.float32) + b3_ref[...][None, :]
            o_ref[...] = z

    return pl.pallas_call(
        body,
        grid=(nk,),
        in_specs=[
            pl.BlockSpec((1, 1, kb), lambda i: (i, 0, 0)),
            pl.BlockSpec((1, kb, 128), lambda i: (i, 0, 0)),
            pl.BlockSpec((128,), lambda i: (0,)),
            pl.BlockSpec((128, 128), lambda i: (0, 0)),
            pl.BlockSpec((128,), lambda i: (0,)),
            pl.BlockSpec((128, 9), lambda i: (0, 0)),
            pl.BlockSpec((9,), lambda i: (0,)),
        ],
        out_specs=pl.BlockSpec((1, 9), lambda i: (0, 0)),
        out_shape=jax.ShapeDtypeStruct((1, 9), jnp.float32),
        scratch_shapes=[pltpu.VMEM((1, 128), jnp.float32)],
    )(xf.reshape(nk, 1, kb), Wf1.reshape(nk, kb, 128), bf1, Wf2, bf2, Wf3,
      bf3)


def kernel(x, edge_index, edge_weight, W1, b1, W2, b2, W3, b3, W4, b4, W5, b5,
           Wf1, bf1, Wf2, bf2, Wf3, bf3):
    ipad = jnp.zeros((EP - E,), jnp.int32)
    fpad = jnp.zeros((EP - E,), jnp.float32)
    src1 = jnp.concatenate([edge_index[0], ipad])
    dst1 = jnp.concatenate([edge_index[1], ipad])
    w1 = jnp.concatenate([edge_weight, fpad])
    zeros1 = jnp.zeros((RPN,), jnp.float32)
    zeros16 = jnp.zeros((RPN, 16), jnp.float32)
    xp = jnp.concatenate([x[:, 0], jnp.zeros((NP - N,), jnp.float32)])

    degp = _sc_degree(src1, w1, zeros1).reshape(NC, NP)
    dinv = _tc_dinv(degp)
    nw1 = _sc_norm(src1, dst1, w1, dinv)

    def lmv1(t):
        return _sc_lmv1(t, src1, dst1, nw1, zeros1).reshape(NC, NP)

    t0 = xp
    t1 = _tc_comb1(lmv1(t0), t0, 1.0, 0.0)
    t2 = _tc_comb1(lmv1(t1), t0, 2.0, -1.0)
    t3 = _tc_comb1(lmv1(t2), t1, 2.0, -1.0)
    h = _tc_layer1([t0, t1, t2, t3], W1, b1)

    def lmv32(t):
        return _sc_lmv32(t[0], t[1], src1, dst1, nw1, zeros16)

    for W, b, fo in ((W2, b2, F), (W3, b3, F), (W4, b4, F), (W5, b5, 4)):
        t0 = h
        t1 = _tc_comb32(lmv32(t0), t0, 1.0, 0.0)
        t2 = _tc_comb32(lmv32(t1), t0, 2.0, -1.0)
        t3 = _tc_comb32(lmv32(t2), t1, 2.0, -1.0)
        h = _tc_layer([t0, t1, t2, t3], W, b, fo)

    xf = h[:N, :].reshape(1, N * 4)
    return _tc_fc(xf, Wf1, bf1, Wf2, bf2, Wf3, bf3)


# speedup vs baseline: 15.3273x; 1.0370x over previous
import functools

import jax
import jax.numpy as jnp
from jax import lax
from jax.experimental import pallas as pl
from jax.experimental.pallas import tpu as pltpu
from jax.experimental.pallas import tpu_sc as plsc

N = 50000
NP = 51200
RPN = NP // 16
E = 1600000
EP = 1622016
NC, NS = 2, 16
NW = NC * NS
EPT = EP // NW
CH = 1056
NBLK = EPT // CH
F = 32

_mesh = plsc.VectorSubcoreMesh(core_axis_name="c", subcore_axis_name="s")
_sc_params = pltpu.CompilerParams(needs_layout_passes=False,
                                 use_tc_tiling_on_sc=False)


def _wid():
    return lax.axis_index("c") * NS + lax.axis_index("s")


@functools.partial(
    pl.kernel,
    out_type=jax.ShapeDtypeStruct((NC * NP,), jnp.float32),
    mesh=_mesh,
    compiler_params=_sc_params,
    scratch_types=[
        pltpu.VMEM((CH,), jnp.int32),
        pltpu.VMEM((CH,), jnp.float32),
        pltpu.VMEM_SHARED((NP,), jnp.float32),
    ],
)
def _sc_degree(src1, w1, zeros1, out, sbuf, wbuf, acc):
    cid = lax.axis_index("c")
    sid = lax.axis_index("s")
    wid = _wid()
    pltpu.sync_copy(zeros1, acc.at[pl.ds(sid * RPN, RPN)])
    plsc.subcore_barrier()

    def blk(i, _):
        e0 = wid * EPT + i * CH
        pltpu.sync_copy(src1.at[pl.ds(e0, CH)], sbuf)
        pltpu.sync_copy(w1.at[pl.ds(e0, CH)], wbuf)
        pltpu.sync_copy(wbuf, acc.at[sbuf], add=True)
        return 0

    lax.fori_loop(0, NBLK, blk, 0)
    plsc.subcore_barrier()
    pltpu.sync_copy(acc.at[pl.ds(sid * RPN, RPN)],
                    out.at[pl.ds(cid * NP + sid * RPN, RPN)])


@functools.partial(
    pl.kernel,
    out_type=jax.ShapeDtypeStruct((EP,), jnp.float32),
    mesh=_mesh,
    compiler_params=_sc_params,
    scratch_types=[
        pltpu.VMEM((CH,), jnp.int32),
        pltpu.VMEM((CH,), jnp.int32),
        pltpu.VMEM((CH,), jnp.float32),
        pltpu.VMEM((CH,), jnp.float32),
        pltpu.VMEM((NP,), jnp.float32),
    ],
)
def _sc_norm(src1, dst1, w1, dinv, out, sbuf, dbuf, wbuf, obuf, tbuf):
    wid = _wid()
    pltpu.sync_copy(dinv, tbuf)

    def blk(i, _):
        e0 = wid * EPT + i * CH
        pltpu.sync_copy(src1.at[pl.ds(e0, CH)], sbuf)
        pltpu.sync_copy(dst1.at[pl.ds(e0, CH)], dbuf)
        pltpu.sync_copy(w1.at[pl.ds(e0, CH)], wbuf)
        for g in range(CH // 16):
            sl = pl.ds(g * 16, 16)
            ds_ = plsc.load_gather(tbuf, [sbuf[sl]])
            dd = plsc.load_gather(tbuf, [dbuf[sl]])
            obuf[sl] = -(ds_ * wbuf[sl] * dd)
        pltpu.sync_copy(obuf, out.at[pl.ds(e0, CH)])
        return 0

    lax.fori_loop(0, NBLK, blk, 0)


@functools.partial(
    pl.kernel,
    out_type=jax.ShapeDtypeStruct((NC * NP,), jnp.float32),
    mesh=_mesh,
    compiler_params=_sc_params,
    scratch_types=[
        pltpu.VMEM((CH,), jnp.int32),
        pltpu.VMEM((CH,), jnp.int32),
        pltpu.VMEM((CH,), jnp.float32),
        pltpu.VMEM((CH,), jnp.float32),
        pltpu.VMEM((NP,), jnp.float32),
        pltpu.VMEM_SHARED((NP,), jnp.float32),
    ],
)
def _sc_lmv1(t, src1, dst1, nw1, zeros1, out, sbuf, dbuf, wbuf, vbuf, tbuf,
             acc):
    cid = lax.axis_index("c")
    sid = lax.axis_index("s")
    wid = _wid()
    pltpu.sync_copy(zeros1, acc.at[pl.ds(sid * RPN, RPN)])
    pltpu.sync_copy(t, tbuf)
    plsc.subcore_barrier()

    def blk(i, _):
        e0 = wid * EPT + i * CH
        pltpu.sync_copy(src1.at[pl.ds(e0, CH)], sbuf)
        pltpu.sync_copy(dst1.at[pl.ds(e0, CH)], dbuf)
        pltpu.sync_copy(nw1.at[pl.ds(e0, CH)], wbuf)
        for g in range(CH // 16):
            sl = pl.ds(g * 16, 16)
            vals = plsc.load_gather(tbuf, [sbuf[sl]])
            vbuf[sl] = vals * wbuf[sl]
        pltpu.sync_copy(vbuf, acc.at[dbuf], add=True)
        return 0

    lax.fori_loop(0, NBLK, blk, 0)
    plsc.subcore_barrier()
    pltpu.sync_copy(acc.at[pl.ds(sid * RPN, RPN)],
                    out.at[pl.ds(cid * NP + sid * RPN, RPN)])


@functools.partial(
    pl.kernel,
    out_type=jax.ShapeDtypeStruct((NC, 2, NP, 16), jnp.float32),
    mesh=_mesh,
    compiler_params=_sc_params,
    scratch_types=[
        pltpu.VMEM((CH,), jnp.int32),
        pltpu.VMEM((CH,), jnp.int32),
        pltpu.VMEM((CH,), jnp.int32),
        pltpu.VMEM((CH,), jnp.int32),
        pltpu.VMEM((CH,), jnp.int32),
        pltpu.VMEM((CH,), jnp.int32),
        pltpu.VMEM((CH,), jnp.float32),
        pltpu.VMEM((CH,), jnp.float32),
        pltpu.VMEM((CH,), jnp.float32),
        pltpu.VMEM((CH, 16), jnp.float32),
        pltpu.VMEM((CH, 16), jnp.float32),
        pltpu.VMEM((CH, 16), jnp.float32),
        pltpu.VMEM_SHARED((NP, 16), jnp.float32),
        pltpu.SemaphoreType.DMA,
        pltpu.SemaphoreType.DMA,
        pltpu.SemaphoreType.DMA,
        pltpu.SemaphoreType.DMA,
        pltpu.SemaphoreType.DMA,
        pltpu.SemaphoreType.DMA,
    ],
)
def _sc_lmv32(t0h, t1h, src1, dst1, nw1, zeros16, out,
              sb0, sb1, sb2, db0, db1, db2, wb0, wb1, wb2, rw0, rw1, rw2,
              acc, sg0, sg1, sg2, ss0, ss1, ss2):
    cid = lax.axis_index("c")
    sid = lax.axis_index("s")
    wid = _wid()
    sb = (sb0, sb1, sb2)
    db = (db0, db1, db2)
    wb = (wb0, wb1, wb2)
    rw = (rw0, rw1, rw2)
    sg = (sg0, sg1, sg2)
    ss = (ss0, ss1, ss2)

    for p, th in ((0, t0h), (1, t1h)):
        pltpu.sync_copy(zeros16, acc.at[pl.ds(sid * RPN, RPN), :])
        plsc.subcore_barrier()

        def load_and_gather(k, slot):
            e0 = wid * EPT + k * CH
            pltpu.sync_copy(src1.at[pl.ds(e0, CH)], sb[slot])
            pltpu.sync_copy(dst1.at[pl.ds(e0, CH)], db[slot])
            pltpu.sync_copy(nw1.at[pl.ds(e0, CH)], wb[slot])
            pltpu.async_copy(th.at[sb[slot]], rw[slot], sg[slot])

        def wait_gather(slot):
            pltpu.make_async_copy(th.at[sb[slot]], rw[slot], sg[slot]).wait()

        def wait_scat(slot):
            pltpu.make_async_copy(th.at[pl.ds(0, CH)], rw[slot],
                                  ss[slot]).wait()

        def mul_scat(slot, wait_prev=True):
            wait_gather(slot)
            rows = rw[slot]
            wbuf = wb[slot]

            @plsc.parallel_loop(0, CH // 16, unroll=2)
            def mul(g):
                wv = wbuf[pl.ds(g * 16, 16)]
                base = g * 16
                for l in range(16):
                    e = base + l
                    rows[e, pl.ds(0, 16)] = rows[e, pl.ds(0, 16)] * wv[l]

            if wait_prev:
                wait_scat((slot + 2) % 3)
            pltpu.async_copy(rows, acc.at[db[slot]], ss[slot], add=True)

        load_and_gather(0, 0)
        load_and_gather(1, 1)
        mul_scat(0, wait_prev=False)
        load_and_gather(2, 2)
        mul_scat(1)
        load_and_gather(3, 0)
        mul_scat(2)
        load_and_gather(4, 1)

        def steady(m, _):
            for b in range(3):
                k = 3 + 3 * m + b
                mul_scat(b)
                load_and_gather(k + 2, (b + 2) % 3)
            return 0

        lax.fori_loop(0, (NBLK - 6) // 3, steady, 0)

        mul_scat((NBLK - 3) % 3)
        load_and_gather(NBLK - 1, (NBLK - 1) % 3)
        mul_scat((NBLK - 2) % 3)
        mul_scat((NBLK - 1) % 3)
        wait_scat((NBLK - 1) % 3)

        plsc.subcore_barrier()
        pltpu.sync_copy(acc.at[pl.ds(sid * RPN, RPN), :],
                        out.at[cid, p, pl.ds(sid * RPN, RPN), :])


def _tc_dinv(degp):
    def body(p_ref, o_ref):
        deg = p_ref[0, :] + p_ref[1, :]
        pos = deg > 0.0
        safe = jnp.where(pos, deg, 1.0)
        o_ref[...] = jnp.where(pos, lax.rsqrt(safe), 0.0)

    return pl.pallas_call(
        body, out_shape=jax.ShapeDtypeStruct((NP,), jnp.float32),
    )(degp)


def _tc_comb1(p, prev, alpha, beta):
    def body(p_ref, v_ref, o_ref):
        s = p_ref[0, :] + p_ref[1, :]
        o_ref[...] = alpha * s + beta * v_ref[...]

    return pl.pallas_call(
        body, out_shape=jax.ShapeDtypeStruct((NP,), jnp.float32),
    )(p, prev)


def _tc_comb32(p, prev, alpha, beta):
    blk = 3200

    def body(p_ref, v_ref, o_ref):
        o_ref[0] = alpha * (p_ref[0, 0] + p_ref[1, 0]) + beta * v_ref[0]
        o_ref[1] = alpha * (p_ref[0, 1] + p_ref[1, 1]) + beta * v_ref[1]

    return pl.pallas_call(
        body,
        grid=(NP // blk,),
        in_specs=[
            pl.BlockSpec((NC, 2, blk, 16), lambda i: (0, 0, i, 0)),
            pl.BlockSpec((2, blk, 16), lambda i: (0, i, 0)),
        ],
        out_specs=pl.BlockSpec((2, blk, 16), lambda i: (0, i, 0)),
        out_shape=jax.ShapeDtypeStruct((2, NP, 16), jnp.float32),
    )(p, prev)


def _tc_layer1(ts, W, b):
    blk = 3200

    def body(t0, t1, t2, t3, w_ref, b_ref, o_ref):
        w = w_ref[:, 0, :]
        acc = t0[...] * w[0][None, :]
        acc += t1[...] * w[1][None, :]
        acc += t2[...] * w[2][None, :]
        acc += t3[...] * w[3][None, :]
        acc = jnp.maximum(acc + b_ref[...][None, :], 0.0)
        o_ref[0] = acc[:, :16]
        o_ref[1] = acc[:, 16:]

    tspec = pl.BlockSpec((blk, 1), lambda i: (i, 0))
    return pl.pallas_call(
        body,
        grid=(NP // blk,),
        in_specs=[tspec, tspec, tspec, tspec,
                  pl.BlockSpec((4, 1, F), lambda i: (0, 0, 0)),
                  pl.BlockSpec((F,), lambda i: (0,))],
        out_specs=pl.BlockSpec((2, blk, 16), lambda i: (0, i, 0)),
        out_shape=jax.ShapeDtypeStruct((2, NP, 16), jnp.float32),
    )(*[t.reshape(NP, 1) for t in ts], W, b)


def _tc_layer(ts, W, b, fo):
    blk = 3200
    halves_out = fo == F

    def body(t0, t1, t2, t3, w_ref, b_ref, o_ref):
        acc = None
        for t_ref, k in ((t0, 0), (t1, 1), (t2, 2), (t3, 3)):
            t = jnp.concatenate([t_ref[0], t_ref[1]], axis=-1)
            d = jnp.dot(t, w_ref[k], preferred_element_type=jnp.float32)
            acc = d if acc is None else acc + d
        acc = jnp.maximum(acc + b_ref[...][None, :], 0.0)
        if halves_out:
            o_ref[0] = acc[:, :16]
            o_ref[1] = acc[:, 16:]
        else:
            o_ref[...] = acc

    tspec = pl.BlockSpec((2, blk, 16), lambda i: (0, i, 0))
    if halves_out:
        out_spec = pl.BlockSpec((2, blk, 16), lambda i: (0, i, 0))
        out_shape = jax.ShapeDtypeStruct((2, NP, 16), jnp.float32)
    else:
        out_spec = pl.BlockSpec((blk, fo), lambda i: (i, 0))
        out_shape = jax.ShapeDtypeStruct((NP, fo), jnp.float32)
    return pl.pallas_call(
        body,
        grid=(NP // blk,),
        in_specs=[tspec, tspec, tspec, tspec,
                  pl.BlockSpec((4, F, fo), lambda i: (0, 0, 0)),
                  pl.BlockSpec((fo,), lambda i: (0,))],
        out_specs=out_spec,
        out_shape=out_shape,
    )(*ts, W, b)


def _tc_fc(xf, Wf1, bf1, Wf2, bf2, Wf3, bf3):
    kb = 2000
    nk = 200000 // kb

    def body(x_ref, w1_ref, b1_ref, w2_ref, b2_ref, w3_ref, b3_ref, o_ref,
             acc):
        i = pl.program_id(0)

        @pl.when(i == 0)
        def _():
            acc[...] = jnp.zeros_like(acc)

        acc[...] += jnp.dot(x_ref[0], w1_ref[0],
                            preferred_element_type=jnp.float32)

        @pl.when(i == nk - 1)
        def _():
            z = acc[...] + b1_ref[...][None, :]
            z = jnp.dot(z, w2_ref[...],
                        preferred_element_type=jnp.float32) + b2_ref[...][None, :]
            z = jnp.dot(z, w3_ref[...],
                        preferred_element_type=jnp.float32) + b3_ref[...][None, :]
            o_ref[...] = z

    return pl.pallas_call(
        body,
        grid=(nk,),
        in_specs=[
            pl.BlockSpec((1, 1, kb), lambda i: (i, 0, 0)),
            pl.BlockSpec((1, kb, 128), lambda i: (i, 0, 0)),
            pl.BlockSpec((128,), lambda i: (0,)),
            pl.BlockSpec((128, 128), lambda i: (0, 0)),
            pl.BlockSpec((128,), lambda i: (0,)),
            pl.BlockSpec((128, 9), lambda i: (0, 0)),
            pl.BlockSpec((9,), lambda i: (0,)),
        ],
        out_specs=pl.BlockSpec((1, 9), lambda i: (0, 0)),
        out_shape=jax.ShapeDtypeStruct((1, 9), jnp.float32),
        scratch_shapes=[pltpu.VMEM((1, 128), jnp.float32)],
    )(xf.reshape(nk, 1, kb), Wf1.reshape(nk, kb, 128), bf1, Wf2, bf2, Wf3,
      bf3)


def kernel(x, edge_index, edge_weight, W1, b1, W2, b2, W3, b3, W4, b4, W5, b5,
           Wf1, bf1, Wf2, bf2, Wf3, bf3):
    ipad = jnp.zeros((EP - E,), jnp.int32)
    fpad = jnp.zeros((EP - E,), jnp.float32)
    src1 = jnp.concatenate([edge_index[0], ipad])
    dst1 = jnp.concatenate([edge_index[1], ipad])
    w1 = jnp.concatenate([edge_weight, fpad])
    zeros1 = jnp.zeros((RPN,), jnp.float32)
    zeros16 = jnp.zeros((RPN, 16), jnp.float32)
    xp = jnp.concatenate([x[:, 0], jnp.zeros((NP - N,), jnp.float32)])

    degp = _sc_degree(src1, w1, zeros1).reshape(NC, NP)
    dinv = _tc_dinv(degp)
    nw1 = _sc_norm(src1, dst1, w1, dinv)

    def lmv1(t):
        return _sc_lmv1(t, src1, dst1, nw1, zeros1).reshape(NC, NP)

    t0 = xp
    t1 = _tc_comb1(lmv1(t0), t0, 1.0, 0.0)
    t2 = _tc_comb1(lmv1(t1), t0, 2.0, -1.0)
    t3 = _tc_comb1(lmv1(t2), t1, 2.0, -1.0)
    h = _tc_layer1([t0, t1, t2, t3], W1, b1)

    def lmv32(t):
        return _sc_lmv32(t[0], t[1], src1, dst1, nw1, zeros16)

    for W, b, fo in ((W2, b2, F), (W3, b3, F), (W4, b4, F), (W5, b5, 4)):
        t0 = h
        t1 = _tc_comb32(lmv32(t0), t0, 1.0, 0.0)
        t2 = _tc_comb32(lmv32(t1), t0, 2.0, -1.0)
        t3 = _tc_comb32(lmv32(t2), t1, 2.0, -1.0)
        h = _tc_layer([t0, t1, t2, t3], W, b, fo)

    xf = h[:N, :].reshape(1, N * 4)
    return _tc_fc(xf, Wf1, bf1, Wf2, bf2, Wf3, bf3)


# speedup vs baseline: 21.3204x; 1.3910x over previous
import functools

import jax
import jax.numpy as jnp
from jax import lax
from jax.experimental import pallas as pl
from jax.experimental.pallas import tpu as pltpu
from jax.experimental.pallas import tpu_sc as plsc

N = 50000
NP = 51200
RPN = NP // 16
E = 1600000
EP = 1622016
NC, NS = 2, 16
NW = NC * NS
EPT = EP // NW
CH = 1056
NBLK = EPT // CH
F = 32
PR = NP // 8

_mesh = plsc.VectorSubcoreMesh(core_axis_name="c", subcore_axis_name="s")
_sc_params = pltpu.CompilerParams(needs_layout_passes=False,
                                 use_tc_tiling_on_sc=False)


def _wid():
    return lax.axis_index("c") * NS + lax.axis_index("s")


@functools.partial(
    pl.kernel,
    out_type=jax.ShapeDtypeStruct((NC * NP,), jnp.float32),
    mesh=_mesh,
    compiler_params=_sc_params,
    scratch_types=[
        pltpu.VMEM((CH,), jnp.int32),
        pltpu.VMEM((CH,), jnp.float32),
        pltpu.VMEM_SHARED((NP,), jnp.float32),
    ],
)
def _sc_degree(src1, w1, zeros1, out, sbuf, wbuf, acc):
    cid = lax.axis_index("c")
    sid = lax.axis_index("s")
    wid = _wid()
    pltpu.sync_copy(zeros1, acc.at[pl.ds(sid * RPN, RPN)])
    plsc.subcore_barrier()

    def blk(i, _):
        e0 = wid * EPT + i * CH
        pltpu.sync_copy(src1.at[pl.ds(e0, CH)], sbuf)
        pltpu.sync_copy(w1.at[pl.ds(e0, CH)], wbuf)
        pltpu.sync_copy(wbuf, acc.at[sbuf], add=True)
        return 0

    lax.fori_loop(0, NBLK, blk, 0)
    plsc.subcore_barrier()
    pltpu.sync_copy(acc.at[pl.ds(sid * RPN, RPN)],
                    out.at[pl.ds(cid * NP + sid * RPN, RPN)])


@functools.partial(
    pl.kernel,
    out_type=jax.ShapeDtypeStruct((EP,), jnp.float32),
    mesh=_mesh,
    compiler_params=_sc_params,
    scratch_types=[
        pltpu.VMEM((CH,), jnp.int32),
        pltpu.VMEM((CH,), jnp.int32),
        pltpu.VMEM((CH,), jnp.float32),
        pltpu.VMEM((CH,), jnp.float32),
        pltpu.VMEM((NP,), jnp.float32),
    ],
)
def _sc_norm(src1, dst1, w1, dinv, out, sbuf, dbuf, wbuf, obuf, tbuf):
    wid = _wid()
    pltpu.sync_copy(dinv, tbuf)

    def blk(i, _):
        e0 = wid * EPT + i * CH
        pltpu.sync_copy(src1.at[pl.ds(e0, CH)], sbuf)
        pltpu.sync_copy(dst1.at[pl.ds(e0, CH)], dbuf)
        pltpu.sync_copy(w1.at[pl.ds(e0, CH)], wbuf)
        for g in range(CH // 16):
            sl = pl.ds(g * 16, 16)
            ds_ = plsc.load_gather(tbuf, [sbuf[sl]])
            dd = plsc.load_gather(tbuf, [dbuf[sl]])
            obuf[sl] = -(ds_ * wbuf[sl] * dd)
        pltpu.sync_copy(obuf, out.at[pl.ds(e0, CH)])
        return 0

    lax.fori_loop(0, NBLK, blk, 0)


@functools.partial(
    pl.kernel,
    out_type=jax.ShapeDtypeStruct((NC * NP,), jnp.float32),
    mesh=_mesh,
    compiler_params=_sc_params,
    scratch_types=[
        pltpu.VMEM((CH,), jnp.int32),
        pltpu.VMEM((CH,), jnp.int32),
        pltpu.VMEM((CH,), jnp.float32),
        pltpu.VMEM((CH,), jnp.float32),
        pltpu.VMEM((NP,), jnp.float32),
        pltpu.VMEM_SHARED((NP,), jnp.float32),
    ],
)
def _sc_lmv1(t, src1, dst1, nw1, zeros1, out, sbuf, dbuf, wbuf, vbuf, tbuf,
             acc):
    cid = lax.axis_index("c")
    sid = lax.axis_index("s")
    wid = _wid()
    pltpu.sync_copy(zeros1, acc.at[pl.ds(sid * RPN, RPN)])
    pltpu.sync_copy(t, tbuf)
    plsc.subcore_barrier()

    def blk(i, _):
        e0 = wid * EPT + i * CH
        pltpu.sync_copy(src1.at[pl.ds(e0, CH)], sbuf)
        pltpu.sync_copy(dst1.at[pl.ds(e0, CH)], dbuf)
        pltpu.sync_copy(nw1.at[pl.ds(e0, CH)], wbuf)
        for g in range(CH // 16):
            sl = pl.ds(g * 16, 16)
            vals = plsc.load_gather(tbuf, [sbuf[sl]])
            vbuf[sl] = vals * wbuf[sl]
        pltpu.sync_copy(vbuf, acc.at[dbuf], add=True)
        return 0

    lax.fori_loop(0, NBLK, blk, 0)
    plsc.subcore_barrier()
    pltpu.sync_copy(acc.at[pl.ds(sid * RPN, RPN)],
                    out.at[pl.ds(cid * NP + sid * RPN, RPN)])


@functools.partial(
    pl.kernel,
    out_type=jax.ShapeDtypeStruct((NC, 2, NP, 16), jnp.float32),
    mesh=_mesh,
    compiler_params=_sc_params,
    scratch_types=[
        pltpu.VMEM((CH,), jnp.int32),
        pltpu.VMEM((CH,), jnp.int32),
        pltpu.VMEM((CH,), jnp.int32),
        pltpu.VMEM((CH,), jnp.int32),
        pltpu.VMEM((CH,), jnp.int32),
        pltpu.VMEM((CH,), jnp.int32),
        pltpu.VMEM((CH,), jnp.float32),
        pltpu.VMEM((CH,), jnp.float32),
        pltpu.VMEM((CH,), jnp.float32),
        pltpu.VMEM((CH, 16), jnp.float32),
        pltpu.VMEM((CH, 16), jnp.float32),
        pltpu.VMEM((CH, 16), jnp.float32),
        pltpu.VMEM_SHARED((NP, 16), jnp.float32),
        pltpu.SemaphoreType.DMA,
        pltpu.SemaphoreType.DMA,
        pltpu.SemaphoreType.DMA,
        pltpu.SemaphoreType.DMA,
        pltpu.SemaphoreType.DMA,
        pltpu.SemaphoreType.DMA,
    ],
)
def _sc_lmv32(t0h, t1h, src1, dst1, nw1, zeros16, out,
              sb0, sb1, sb2, db0, db1, db2, wb0, wb1, wb2, rw0, rw1, rw2,
              acc, sg0, sg1, sg2, ss0, ss1, ss2):
    cid = lax.axis_index("c")
    sid = lax.axis_index("s")
    wid = _wid()
    sb = (sb0, sb1, sb2)
    db = (db0, db1, db2)
    wb = (wb0, wb1, wb2)
    rw = (rw0, rw1, rw2)
    sg = (sg0, sg1, sg2)
    ss = (ss0, ss1, ss2)

    for p, th in ((0, t0h), (1, t1h)):
        pltpu.sync_copy(zeros16, acc.at[pl.ds(sid * RPN, RPN), :])
        plsc.subcore_barrier()

        def load_and_gather(k, slot):
            e0 = wid * EPT + k * CH
            pltpu.sync_copy(src1.at[pl.ds(e0, CH)], sb[slot])
            pltpu.sync_copy(dst1.at[pl.ds(e0, CH)], db[slot])
            pltpu.sync_copy(nw1.at[pl.ds(e0, CH)], wb[slot])
            pltpu.async_copy(th.at[sb[slot]], rw[slot], sg[slot])

        def wait_gather(slot):
            pltpu.make_async_copy(th.at[sb[slot]], rw[slot], sg[slot]).wait()

        def wait_scat(slot):
            pltpu.make_async_copy(th.at[pl.ds(0, CH)], rw[slot],
                                  ss[slot]).wait()

        def mul_scat(slot, wait_prev=True):
            wait_gather(slot)
            rows = rw[slot]
            wbuf = wb[slot]

            @plsc.parallel_loop(0, CH // 16, unroll=2)
            def mul(g):
                wv = wbuf[pl.ds(g * 16, 16)]
                base = g * 16
                for l in range(16):
                    e = base + l
                    rows[e, pl.ds(0, 16)] = rows[e, pl.ds(0, 16)] * wv[l]

            if wait_prev:
                wait_scat((slot + 2) % 3)
            pltpu.async_copy(rows, acc.at[db[slot]], ss[slot], add=True)

        load_and_gather(0, 0)
        load_and_gather(1, 1)
        mul_scat(0, wait_prev=False)
        load_and_gather(2, 2)
        mul_scat(1)
        load_and_gather(3, 0)
        mul_scat(2)
        load_and_gather(4, 1)

        def steady(m, _):
            for b in range(3):
                k = 3 + 3 * m + b
                mul_scat(b)
                load_and_gather(k + 2, (b + 2) % 3)
            return 0

        lax.fori_loop(0, (NBLK - 6) // 3, steady, 0)

        mul_scat((NBLK - 3) % 3)
        load_and_gather(NBLK - 1, (NBLK - 1) % 3)
        mul_scat((NBLK - 2) % 3)
        mul_scat((NBLK - 1) % 3)
        wait_scat((NBLK - 1) % 3)

        plsc.subcore_barrier()
        pltpu.sync_copy(acc.at[pl.ds(sid * RPN, RPN), :],
                        out.at[cid, p, pl.ds(sid * RPN, RPN), :])


_CCH = 800


def _make_sc_comb(alpha, beta):
    @functools.partial(
        pl.kernel,
        out_type=[jax.ShapeDtypeStruct((NP, 16), jnp.float32)] * 2,
        mesh=_mesh,
        compiler_params=_sc_params,
        scratch_types=[
            pltpu.VMEM((_CCH, 16), jnp.float32),
            pltpu.VMEM((_CCH, 16), jnp.float32),
            pltpu.VMEM((_CCH, 16), jnp.float32),
            pltpu.VMEM((_CCH, 16), jnp.float32),
        ],
    )
    def comb(p, prev0, prev1, o0, o1, va, vb, vc, vo):
        sid = lax.axis_index("s")
        cid = lax.axis_index("c")
        half = RPN // 2
        base = sid * RPN + cid * half
        prevs = (prev0, prev1)
        outs = (o0, o1)
        for h in range(2):
            def chunk(c, _):
                r0 = base + c * _CCH
                pltpu.sync_copy(p.at[0, h, pl.ds(r0, _CCH), :], va)
                pltpu.sync_copy(p.at[1, h, pl.ds(r0, _CCH), :], vb)
                if beta != 0.0:
                    pltpu.sync_copy(prevs[h].at[pl.ds(r0, _CCH), :], vc)

                @plsc.parallel_loop(0, _CCH, unroll=8)
                def cc(r):
                    sl = pl.ds(0, 16)
                    v = alpha * (va[r, sl] + vb[r, sl])
                    if beta != 0.0:
                        v = v + beta * vc[r, sl]
                    vo[r, sl] = v

                pltpu.sync_copy(vo, outs[h].at[pl.ds(r0, _CCH), :])
                return 0

            lax.fori_loop(0, half // _CCH, chunk, 0)

    return comb


_sc_comb_first = _make_sc_comb(1.0, 0.0)
_sc_comb_rec = _make_sc_comb(2.0, -1.0)


@functools.partial(
    pl.kernel,
    out_type=[jax.ShapeDtypeStruct((NP, 16), jnp.float32)] * 2,
    mesh=_mesh,
    compiler_params=_sc_params,
    scratch_types=[
        pltpu.VMEM((RPN // 2,), jnp.float32),
        pltpu.VMEM((RPN // 2,), jnp.float32),
        pltpu.VMEM((RPN // 2,), jnp.float32),
        pltpu.VMEM((RPN // 2,), jnp.float32),
        pltpu.VMEM((RPN // 2, 16), jnp.float32),
        pltpu.VMEM((RPN // 2, 16), jnp.float32),
        pltpu.VMEM((4, 1, F), jnp.float32),
        pltpu.VMEM((F,), jnp.float32),
    ],
)
def _sc_layer1(t0, t1, t2, t3, W, b, o0, o1, tb0, tb1, tb2, tb3, ob0, ob1,
               wbuf, bbuf):
    sid = lax.axis_index("s")
    cid = lax.axis_index("c")
    half = RPN // 2
    base = sid * RPN + cid * half
    pltpu.sync_copy(W, wbuf)
    pltpu.sync_copy(b, bbuf)
    for tin, tb in ((t0, tb0), (t1, tb1), (t2, tb2), (t3, tb3)):
        pltpu.sync_copy(tin.at[pl.ds(base, half)], tb)
    w0 = [wbuf[k, 0, pl.ds(0, 16)] for k in range(4)]
    w1 = [wbuf[k, 0, pl.ds(16, 16)] for k in range(4)]
    b0 = bbuf[pl.ds(0, 16)]
    b1 = bbuf[pl.ds(16, 16)]
    tbs = (tb0, tb1, tb2, tb3)

    @plsc.parallel_loop(0, half // 16, unroll=2)
    def grp(g):
        tv = [tbs[k][pl.ds(g * 16, 16)] for k in range(4)]
        for l in range(16):
            r = g * 16 + l
            acc0 = b0
            acc1 = b1
            for k in range(4):
                acc0 = acc0 + tv[k][l] * w0[k]
                acc1 = acc1 + tv[k][l] * w1[k]
            ob0[r, pl.ds(0, 16)] = jnp.maximum(acc0, 0.0)
            ob1[r, pl.ds(0, 16)] = jnp.maximum(acc1, 0.0)

    pltpu.sync_copy(ob0, o0.at[pl.ds(base, half), :])
    pltpu.sync_copy(ob1, o1.at[pl.ds(base, half), :])


def _tc_dinv(degp):
    def body(p_ref, o_ref):
        deg = p_ref[0, :] + p_ref[1, :]
        pos = deg > 0.0
        safe = jnp.where(pos, deg, 1.0)
        o_ref[...] = jnp.where(pos, lax.rsqrt(safe), 0.0)

    return pl.pallas_call(
        body, out_shape=jax.ShapeDtypeStruct((NP,), jnp.float32),
    )(degp)


def _tc_comb1(p, prev, alpha, beta):
    def body(p_ref, v_ref, o_ref):
        s = p_ref[0, :] + p_ref[1, :]
        o_ref[...] = alpha * s + beta * v_ref[...]

    return pl.pallas_call(
        body, out_shape=jax.ShapeDtypeStruct((NP,), jnp.float32),
    )(p, prev)


def _tc_layer_packed(ts, Bd, bv0, bv1, two_halves):
    blk = PR // 10
    nh = 2 if two_halves else 1

    def body(h00, h01, h10, h11, h20, h21, h30, h31, w_ref, b0_ref, b1_ref,
             *outs):
        hs = (h00, h01, h10, h11, h20, h21, h30, h31)
        for oh in range(nh):
            acc = None
            for k in range(4):
                for ih in range(2):
                    d = jnp.dot(hs[k * 2 + ih][...], w_ref[k, ih, oh],
                                preferred_element_type=jnp.float32)
                    acc = d if acc is None else acc + d
            bv = (b0_ref, b1_ref)[oh]
            outs[oh][...] = jnp.maximum(acc + bv[...][None, :], 0.0)

    pspec = pl.BlockSpec((blk, 128), lambda i: (i, 0))
    res = pl.pallas_call(
        body,
        grid=(PR // blk,),
        in_specs=[pspec] * 8 + [
            pl.BlockSpec((4, 2, nh, 128, 128), lambda i: (0, 0, 0, 0, 0)),
            pl.BlockSpec((128,), lambda i: (0,)),
            pl.BlockSpec((128,), lambda i: (0,)),
        ],
        out_specs=[pspec] * nh,
        out_shape=[jax.ShapeDtypeStruct((PR, 128), jnp.float32)] * nh,
    )(*[jnp.reshape(hh, (PR, 128)) for t in ts for hh in t], Bd, bv0, bv1)
    return res


def _tc_fc(xf, Wf1, bf1, Wf2, bf2, Wf3, bf3):
    kb = 2000
    nk = 200000 // kb

    def body(x_ref, w1_ref, b1_ref, w2_ref, b2_ref, w3_ref, b3_ref, o_ref,
             acc):
        i = pl.program_id(0)

        @pl.when(i == 0)
        def _():
            acc[...] = jnp.zeros_like(acc)

        acc[...] += jnp.dot(x_ref[0], w1_ref[0],
                            preferred_element_type=jnp.float32)

        @pl.when(i == nk - 1)
        def _():
            z = acc[...] + b1_ref[...][None, :]
            z = jnp.dot(z, w2_ref[...],
                        preferred_element_type=jnp.float32) + b2_ref[...][None, :]
            z = jnp.dot(z, w3_ref[...],
                        preferred_element_type=jnp.float32) + b3_ref[...][None, :]
            o_ref[...] = z

    return pl.pallas_call(
        body,
        grid=(nk,),
        in_specs=[
            pl.BlockSpec((1, 1, kb), lambda i: (i, 0, 0)),
            pl.BlockSpec((1, kb, 128), lambda i: (i, 0, 0)),
            pl.BlockSpec((128,), lambda i: (0,)),
            pl.BlockSpec((128, 128), lambda i: (0, 0)),
            pl.BlockSpec((128,), lambda i: (0,)),
            pl.BlockSpec((128, 9), lambda i: (0, 0)),
            pl.BlockSpec((9,), lambda i: (0,)),
        ],
        out_specs=pl.BlockSpec((1, 9), lambda i: (0, 0)),
        out_shape=jax.ShapeDtypeStruct((1, 9), jnp.float32),
        scratch_shapes=[pltpu.VMEM((1, 128), jnp.float32)],
    )(xf.reshape(nk, 1, kb), Wf1.reshape(nk, kb, 128), bf1, Wf2, bf2, Wf3,
      bf3)


def _blockdiag(W, fo_halves):
    fo = W.shape[-1]
    Wp = jnp.concatenate(
        [W, jnp.zeros((4, F, 16 * fo_halves - fo), W.dtype)], axis=-1)
    Ws = Wp.reshape(4, 2, 16, fo_halves, 16).transpose(0, 1, 3, 2, 4)
    eye8 = jnp.eye(8, dtype=W.dtype)
    Bd = (eye8[None, None, None, :, None, :, None]
          * Ws[:, :, :, None, :, None, :])
    return Bd.reshape(4, 2, fo_halves, 128, 128)


def _bvec(b, h):
    bp = jnp.concatenate([b, jnp.zeros((F - b.shape[0],), b.dtype)])
    return jnp.tile(bp[h * 16:(h + 1) * 16], 8)


def kernel(x, edge_index, edge_weight, W1, b1, W2, b2, W3, b3, W4, b4, W5, b5,
           Wf1, bf1, Wf2, bf2, Wf3, bf3):
    ipad = jnp.zeros((EP - E,), jnp.int32)
    fpad = jnp.zeros((EP - E,), jnp.float32)
    src1 = jnp.concatenate([edge_index[0], ipad])
    dst1 = jnp.concatenate([edge_index[1], ipad])
    w1 = jnp.concatenate([edge_weight, fpad])
    zeros1 = jnp.zeros((RPN,), jnp.float32)
    zeros16 = jnp.zeros((RPN, 16), jnp.float32)
    xp = jnp.concatenate([x[:, 0], jnp.zeros((NP - N,), jnp.float32)])

    degp = _sc_degree(src1, w1, zeros1).reshape(NC, NP)
    dinv = _tc_dinv(degp)
    nw1 = _sc_norm(src1, dst1, w1, dinv)

    def lmv1(t):
        return _sc_lmv1(t, src1, dst1, nw1, zeros1).reshape(NC, NP)

    t0 = xp
    t1 = _tc_comb1(lmv1(t0), t0, 1.0, 0.0)
    t2 = _tc_comb1(lmv1(t1), t0, 2.0, -1.0)
    t3 = _tc_comb1(lmv1(t2), t1, 2.0, -1.0)
    h = _sc_layer1(t0, t1, t2, t3, W1, b1)

    def lmv32(t):
        return _sc_lmv32(t[0], t[1], src1, dst1, nw1, zeros16)

    h5 = None
    for W, b, fo in ((W2, b2, F), (W3, b3, F), (W4, b4, F), (W5, b5, 4)):
        t0 = h
        t1 = _sc_comb_first(lmv32(t0), t0[0], t0[1])
        t2 = _sc_comb_rec(lmv32(t1), t0[0], t0[1])
        t3 = _sc_comb_rec(lmv32(t2), t1[0], t1[1])
        two = fo == F
        res = _tc_layer_packed([t0, t1, t2, t3],
                               _blockdiag(W, 2 if two else 1),
                               _bvec(b, 0), _bvec(b, 1), two)
        if two:
            h = (jnp.reshape(res[0], (NP, 16)), jnp.reshape(res[1], (NP, 16)))
        else:
            h5 = jnp.reshape(res[0], (NP, 16))

    xf = h5[:N, :4].reshape(1, N * 4)
    return _tc_fc(xf, Wf1, bf1, Wf2, bf2, Wf3, bf3)
